# Initial kernel scaffold; baseline (speedup 1.0000x reference)
import jax
import jax.numpy as jnp
from jax.experimental import pallas as pl


def kernel(tokenIndex, weights):
    raise NotImplementedError("write your pallas kernel here")


# speedup vs baseline: 1.8459x; 1.8459x over previous
import functools

import jax
import jax.numpy as jnp
from jax import lax
from jax.experimental import pallas as pl
from jax.experimental.pallas import tpu as pltpu
from jax.experimental.pallas import tpu_sc as plsc

_LANES = 128
_K = 8


def _gather_sc(weights, idx2d):
    n_rows, lanes = idx2d.shape
    d = weights.shape[1]
    info = plsc.get_sparse_core_info()
    nc, ns = info.num_cores, info.num_subcores
    nw = nc * ns
    rows_per_w = n_rows // nw
    steps = rows_per_w // _K

    mesh = plsc.VectorSubcoreMesh(core_axis_name="c", subcore_axis_name="s")

    @functools.partial(
        pl.kernel,
        mesh=mesh,
        compiler_params=pltpu.CompilerParams(use_tc_tiling_on_sc=False),
        out_type=jax.ShapeDtypeStruct((n_rows, lanes, d), jnp.float32),
        scratch_types=[
            pltpu.VMEM((_K, lanes), jnp.int32),
            pltpu.VMEM((_K, lanes, d), jnp.float32),
            pltpu.SemaphoreType.DMA,
        ],
    )
    def k(table_hbm, idx_hbm, out_hbm, idx_v, rows_v, sem):
        wid = lax.axis_index("s") * nc + lax.axis_index("c")
        base = wid * rows_per_w

        def step(g, carry):
            row0 = base + g * _K
            pltpu.sync_copy(idx_hbm.at[pl.ds(row0, _K)], idx_v)
            cps = [
                pltpu.make_async_copy(table_hbm.at[idx_v.at[j]], rows_v.at[j], sem)
                for j in range(_K)
            ]
            for cp in cps:
                cp.start()
            for cp in cps:
                cp.wait()
            pltpu.sync_copy(rows_v, out_hbm.at[pl.ds(row0, _K)])
            return carry

        lax.fori_loop(0, steps, step, 0)

    return k(weights, idx2d)


def kernel(tokenIndex, weights):
    b, h = tokenIndex.shape
    d = weights.shape[1]
    idx2d = tokenIndex.reshape(-1, _LANES)
    out = _gather_sc(weights, idx2d)
    return out.reshape(b, h, d)


# speedup vs baseline: 1.8740x; 1.0152x over previous
import functools

import jax
import jax.numpy as jnp
from jax import lax
from jax.experimental import pallas as pl
from jax.experimental.pallas import tpu as pltpu
from jax.experimental.pallas import tpu_sc as plsc

_LANES = 128
_K = 5


def _gather_sc(weights, idx2d):
    n_rows, lanes = idx2d.shape
    d = weights.shape[1]
    info = plsc.get_sparse_core_info()
    nc, ns = info.num_cores, info.num_subcores
    nw = nc * ns
    rows_per_w = n_rows // nw
    steps = rows_per_w // _K
    n2 = steps // 2

    mesh = plsc.VectorSubcoreMesh(core_axis_name="c", subcore_axis_name="s")

    @functools.partial(
        pl.kernel,
        mesh=mesh,
        compiler_params=pltpu.CompilerParams(use_tc_tiling_on_sc=False),
        out_type=jax.ShapeDtypeStruct((n_rows, lanes, d), jnp.float32),
        scratch_types=[
            pltpu.VMEM((rows_per_w, lanes), jnp.int32),
            pltpu.VMEM((_K, lanes, d), jnp.float32),
            pltpu.VMEM((_K, lanes, d), jnp.float32),
            pltpu.SemaphoreType.DMA,
            pltpu.SemaphoreType.DMA,
            pltpu.SemaphoreType.DMA,
            pltpu.SemaphoreType.DMA,
        ],
    )
    def k(table_hbm, idx_hbm, out_hbm, idx_all, rows0, rows1, sg0, sg1, sw0, sw1):
        wid = lax.axis_index("s") * nc + lax.axis_index("c")
        base = wid * rows_per_w
        rows = (rows0, rows1)
        sg = (sg0, sg1)
        sw = (sw0, sw1)

        pltpu.sync_copy(idx_hbm.at[pl.ds(base, rows_per_w)], idx_all)

        def fire(g, p):
            for j in range(_K):
                pltpu.make_async_copy(
                    table_hbm.at[idx_all.at[g * _K + j]], rows[p].at[j], sg[p]
                ).start()

        def drain(p):
            for j in range(_K):
                pltpu.make_async_copy(
                    table_hbm.at[pl.ds(0, lanes)], rows[p].at[j], sg[p]
                ).wait()

        def writeback(g, p):
            pltpu.make_async_copy(
                rows[p], out_hbm.at[pl.ds(base + g * _K, _K)], sw[p]
            ).start()

        def wait_wb(p):
            pltpu.make_async_copy(
                rows[p], out_hbm.at[pl.ds(base, _K)], sw[p]
            ).wait()

        fire(0, 0)

        def body(i, carry):
            g0 = 2 * i
            g1 = g0 + 1

            @pl.when(i > 0)
            def _():
                wait_wb(1)

            fire(g1, 1)
            drain(0)
            writeback(g0, 0)

            @pl.when(i < n2 - 1)
            def _():
                wait_wb(0)
                fire(g0 + 2, 0)

            drain(1)
            writeback(g1, 1)
            return carry

        lax.fori_loop(0, n2, body, 0)
        wait_wb(0)
        wait_wb(1)

    return k(weights, idx2d)


def kernel(tokenIndex, weights):
    b, h = tokenIndex.shape
    d = weights.shape[1]
    idx2d = tokenIndex.reshape(-1, _LANES)
    out = _gather_sc(weights, idx2d)
    return out.reshape(b, h, d)
